# baseline (speedup 1.0000x reference)
import functools

import jax
import jax.numpy as jnp
import numpy as np
from jax import lax
from jax.experimental import pallas as pl
from jax.experimental.pallas import tpu as pltpu
from jax.experimental.pallas import tpu_sc as plsc

N = 10000
E = 320000
D = 128

NUM_CORES = 2
NUM_SUBCORES = 16
SUB = 80
STAGE = 2 * SUB
NSTAGES = E // STAGE
NST = 62
ROWS_MAIN = 624
MM_BLOCK = 1000

_blk = np.concatenate([np.arange(0, 32, 2), np.arange(1, 32, 2)])
UNPACK_PERM = np.concatenate([b * 32 + _blk for b in range(D // 32)])


def _fin_body(a_ref, b_ref, w_ref, o_ref):
    x = a_ref[...] + b_ref[...]
    y = lax.dot_general(x, w_ref[...], (((1,), (1,)), ((), ())),
                        preferred_element_type=jnp.float32)
    o_ref[...] = jnp.maximum(y, 0.0)


def _finish(a, b, w):
    return pl.pallas_call(
        _fin_body,
        grid=(N // MM_BLOCK,),
        in_specs=[
            pl.BlockSpec((MM_BLOCK, D), lambda i: (i, 0)),
            pl.BlockSpec((MM_BLOCK, D), lambda i: (i, 0)),
            pl.BlockSpec((D, D), lambda i: (0, 0)),
        ],
        out_specs=pl.BlockSpec((MM_BLOCK, D), lambda i: (i, 0)),
        out_shape=jax.ShapeDtypeStruct((N, D), jnp.float32),
    )(a, b, w)


@functools.partial(
    pl.kernel,
    mesh=plsc.VectorSubcoreMesh(core_axis_name="c", subcore_axis_name="s"),
    out_type=jax.ShapeDtypeStruct((NUM_CORES, N, D), jnp.float32),
    compiler_params=pltpu.CompilerParams(use_tc_tiling_on_sc=False,
                                         needs_layout_passes=False),
    scratch_types=[
        pltpu.VMEM((2, SUB), jnp.int32),
        pltpu.VMEM((2, SUB), jnp.int32),
        pltpu.VMEM((2, SUB), jnp.int32),
        pltpu.VMEM((2, SUB), jnp.int32),
        pltpu.VMEM((STAGE // 8, D), jnp.float32),
        pltpu.VMEM((STAGE // 8, D), jnp.float32),
        pltpu.VMEM((STAGE, D), jnp.bfloat16),
        pltpu.VMEM((STAGE, D), jnp.bfloat16),
        pltpu.VMEM((STAGE, D), jnp.float32),
        pltpu.VMEM_SHARED((N, D), jnp.float32),
        pltpu.SemaphoreType.DMA,
        pltpu.SemaphoreType.DMA,
        pltpu.SemaphoreType.DMA,
        pltpu.SemaphoreType.DMA,
        pltpu.SemaphoreType.DMA,
        pltpu.SemaphoreType.DMA,
        pltpu.SemaphoreType.DMA,
        pltpu.SemaphoreType.DMA,
    ],
)
def _spmm_sc(seq_hbm, col_hbm, row_hbm, w_hbm, part_hbm,
             col_a, col_b, row_a, row_b, w_a, w_b, bf_a, bf_b, msg_f, acc,
             sem_c0, sem_c1, sem_r0, sem_r1, sem_w0, sem_w1,
             sem_g0, sem_g1):
    cols = (col_a, col_b)
    rows = (row_a, row_b)
    ws = (w_a, w_b)
    bfs = (bf_a, bf_b)
    c = lax.axis_index("c")
    s = lax.axis_index("s")
    wid = c * NUM_SUBCORES + s
    sems_c = (sem_c0, sem_c1)
    sems_r = (sem_r0, sem_r1)
    sems_w = (sem_w0, sem_w1)
    sems_g = (sem_g0, sem_g1)

    pair = wid // 2
    within = wid % 2
    gbase = pair * 125 + within * 63

    def drain_idx(dst, sem):
        pltpu.make_async_copy(col_hbm.at[0], dst, sem).wait()

    def drain_w(slot):
        pltpu.make_async_copy(w_hbm.at[0], ws[slot], sems_w[slot]).wait()

    def drain_g(slot):
        for j in range(2):
            pltpu.make_async_copy(seq_hbm.at[cols[slot].at[j]],
                                  bfs[slot].at[pl.ds(j * SUB, SUB)],
                                  sems_g[slot]).wait()

    def fire_gathers(slot):
        for j in range(2):
            pltpu.async_copy(seq_hbm.at[cols[slot].at[j]],
                             bfs[slot].at[pl.ds(j * SUB, SUB)],
                             sems_g[slot])

    def scale(slot):
        def _body(kk, carry):
            base = kk * 8
            for t in range(8):
                wk = ws[slot][kk, pl.ds(t * 16, 16)]
                k = base + t
                for j in range(D // 32):
                    xb = bfs[slot][k, pl.ds(j * 32, 32)]
                    u0, u1 = plsc.unpack(xb,
                                         format=plsc.PackFormat.INTERLEAVED)
                    msg_f[k, pl.ds(j * 32, 16)] = u0 * wk
                    msg_f[k, pl.ds(j * 32 + 16, 16)] = u1 * wk
            return carry

        lax.fori_loop(0, STAGE // 8, _body, 0)

    zero16 = jnp.zeros((16,), jnp.float32)

    def _zero_row(k, carry):
        for j in range(D // 16):
            msg_f[k, pl.ds(j * 16, 16)] = zero16
        return carry

    lax.fori_loop(0, STAGE, _zero_row, 0)
    r0 = s * ROWS_MAIN
    copy_sizes = (STAGE, STAGE, STAGE, 144)
    copy_sizes_last = (STAGE, STAGE, STAGE, STAGE)

    @pl.when(s == NUM_SUBCORES - 1)
    def _():
        off = 0
        for sz in copy_sizes_last:
            pltpu.sync_copy(msg_f.at[pl.ds(0, sz)],
                            acc.at[pl.ds(r0 + off, sz)])
            off += sz

    @pl.when(s != NUM_SUBCORES - 1)
    def _():
        off = 0
        for sz in copy_sizes:
            pltpu.sync_copy(msg_f.at[pl.ds(0, sz)],
                            acc.at[pl.ds(r0 + off, sz)])
            off += sz

    plsc.subcore_barrier()

    pltpu.async_copy(col_hbm.at[gbase], col_a, sem_c0)
    pltpu.async_copy(row_hbm.at[gbase], row_a, sem_r0)
    pltpu.async_copy(w_hbm.at[gbase], w_a, sem_w0)
    pltpu.async_copy(col_hbm.at[gbase + 1], col_b, sem_c1)
    pltpu.async_copy(w_hbm.at[gbase + 1], w_b, sem_w1)
    drain_idx(col_a, sem_c0)
    fire_gathers(0)

    def _iter(t2, carry):
        for parity in range(2):
            p = parity
            q = 1 - p
            t = 2 * t2 + parity
            g = gbase + t

            def _fut1_ops():
                drain_idx(cols[q], sems_c[q])
                pltpu.async_copy(row_hbm.at[g + 1], rows[q],
                                 sems_r[q])
                fire_gathers(q)

            def _fut2_fire_col():
                pltpu.async_copy(col_hbm.at[g + 2], cols[p], sems_c[p])

            def _fut2_fire_w():
                pltpu.async_copy(w_hbm.at[g + 2], ws[p], sems_w[p])

            if parity == 0:
                _fut1_ops()
            else:
                @pl.when(t2 < 30)
                def _():
                    _fut1_ops()

            drain_g(p)

            @pl.when(t2 < 30)
            def _():
                _fut2_fire_col()

            drain_w(p)
            scale(p)
            drain_idx(rows[p], sems_r[p])
            for j in range(2):
                pltpu.sync_copy(msg_f.at[pl.ds(j * SUB, SUB)],
                                acc.at[rows[p].at[j]], add=True)

            @pl.when(t2 < 30)
            def _():
                _fut2_fire_w()
        return carry

    lax.fori_loop(0, NST // 2, _iter, 0)

    @pl.when(within == 0)
    def _():
        g = gbase + NST
        pltpu.sync_copy(col_hbm.at[g], col_a)
        pltpu.sync_copy(row_hbm.at[g], row_a)
        pltpu.sync_copy(w_hbm.at[g], w_a)
        fire_gathers(0)
        drain_g(0)
        scale(0)
        for j in range(2):
            pltpu.sync_copy(msg_f.at[pl.ds(j * SUB, SUB)],
                            acc.at[row_a.at[j]], add=True)

    plsc.subcore_barrier()

    @pl.when(s == NUM_SUBCORES - 1)
    def _():
        off = 0
        for sz in copy_sizes_last:
            pltpu.sync_copy(acc.at[pl.ds(r0 + off, sz)],
                            msg_f.at[pl.ds(0, sz)])
            pltpu.sync_copy(msg_f.at[pl.ds(0, sz)],
                            part_hbm.at[c, pl.ds(r0 + off, sz)])
            off += sz

    @pl.when(s != NUM_SUBCORES - 1)
    def _():
        off = 0
        for sz in copy_sizes:
            pltpu.sync_copy(acc.at[pl.ds(r0 + off, sz)],
                            msg_f.at[pl.ds(0, sz)])
            pltpu.sync_copy(msg_f.at[pl.ds(0, sz)],
                            part_hbm.at[c, pl.ds(r0 + off, sz)])
            off += sz


def kernel(seq, edge_index, edge_weight, W):
    col = edge_index[1].astype(jnp.int32).reshape(NSTAGES, 2, SUB)
    row = edge_index[0].astype(jnp.int32).reshape(NSTAGES, 2, SUB)
    wb = jnp.repeat(edge_weight.reshape(E // 8, 8), 16,
                    axis=-1).reshape(NSTAGES, STAGE // 8, D)
    seq_bf = seq.astype(jnp.bfloat16)
    w_perm = W[:, UNPACK_PERM]
    part = _spmm_sc(seq_bf, col, row, wb)
    return _finish(part[0], part[1], w_perm)
